# baseline (speedup 1.0000x reference)
import functools

import jax
import jax.numpy as jnp
from jax import lax
from jax.experimental import pallas as pl
from jax.experimental.pallas import tpu as pltpu
from jax.experimental.pallas import tpu_sc as plsc

NC, NS = 2, 16
NW = NC * NS
_F32 = jnp.float32
_HI = lax.Precision.HIGHEST


def _gather_body(x_hbm, ii_hbm, jj_hbm, xi_hbm, xj_hbm,
                 iv, jv, ri, rj, s1, s2, *, epw, chunk):
    c = lax.axis_index("c")
    s = lax.axis_index("s")
    wid = s * NC + c

    def body(k, carry):
        base = wid * epw + k * chunk
        a = pltpu.async_copy(ii_hbm.at[pl.ds(base, chunk)], iv, s1)
        b = pltpu.async_copy(jj_hbm.at[pl.ds(base, chunk)], jv, s2)
        a.wait()
        b.wait()
        g1 = pltpu.async_copy(x_hbm.at[iv], ri, s1)
        g2 = pltpu.async_copy(x_hbm.at[jv], rj, s2)
        g1.wait()
        g2.wait()
        w1 = pltpu.async_copy(ri, xi_hbm.at[pl.ds(base, chunk)], s1)
        w2 = pltpu.async_copy(rj, xj_hbm.at[pl.ds(base, chunk)], s2)
        w1.wait()
        w2.wait()
        return carry

    lax.fori_loop(0, epw // chunk, body, 0)


def _scatter_body(z_hbm, msg_hbm, ii_hbm, out_hbm, iv, mv, acc,
                  *, n, hid, epw, chunk):
    c = lax.axis_index("c")
    s = lax.axis_index("s")
    wid = s * NC + c
    stripe = n // NS

    pltpu.sync_copy(z_hbm.at[pl.ds(s * stripe, stripe)],
                    acc.at[pl.ds(s * stripe, stripe)])
    plsc.subcore_barrier()

    def body(k, carry):
        base = wid * epw + k * chunk
        pltpu.sync_copy(ii_hbm.at[pl.ds(base, chunk)], iv)
        pltpu.sync_copy(msg_hbm.at[pl.ds(base, chunk)], mv)
        pltpu.sync_copy(mv, acc.at[iv], add=True)
        return carry

    lax.fori_loop(0, epw // chunk, body, 0)
    plsc.subcore_barrier()
    pltpu.sync_copy(acc.at[pl.ds(s * stripe, stripe)],
                    out_hbm.at[pl.ds(c * n + s * stripe, stripe)])


def _mlp_body(xi, xj, ea, w1a, w1b, w1c, b1, w2t, b2, out):
    h = (jnp.dot(xi[...], w1a[...], preferred_element_type=_F32)
         + jnp.dot(xj[...], w1b[...], preferred_element_type=_F32)
         + jnp.dot(ea[...], w1c[...], preferred_element_type=_F32)
         + b1[...])
    h = jnp.maximum(h, 0.0)
    msg8 = (jnp.dot(h, w2t[...], preferred_element_type=_F32)
            + b2[...])
    br = msg8.shape[0]
    out[...] = (msg8.reshape(br // 8, 8, 2, 128)
                .swapaxes(1, 2)
                .reshape(2 * br, 128))


def _softplus(z):
    return jnp.maximum(z, 0.0) + jnp.log(1.0 + jnp.exp(-jnp.abs(z)))


def _node_body(xr, pr, wbx, wbh, wmx, wmh, wsx, wsh,
               bc, bmu, bsig, highr, inv_out, ord_out):
    xpp = pr[0] + pr[1]
    xv = xr[...]
    zc = (jnp.dot(xv, wbx[...], preferred_element_type=_F32, precision=_HI)
          + jnp.dot(xpp, wbh[...], preferred_element_type=_F32, precision=_HI)
          + bc[...] + 1e-10)
    conc = _softplus(zc)
    total = jnp.sum(conc)
    inv_out[...] = conc / (total + 1e-20)
    zmu = (jnp.dot(xv, wmx[...], preferred_element_type=_F32, precision=_HI)
           + jnp.dot(xpp, wmh[...], preferred_element_type=_F32, precision=_HI)
           + bmu[...] + 1e-20)
    a = _softplus(zmu) + 1e-20
    zsg = (jnp.dot(xv, wsx[...], preferred_element_type=_F32, precision=_HI)
           + jnp.dot(xpp, wsh[...], preferred_element_type=_F32, precision=_HI)
           + bsig[...] + 1e-20)
    b = _softplus(zsg) + 1e-20
    ord_out[...] = a / (a + b) * highr[...]


def kernel(x, edge_index, edge_attr, W1, b1, W2, b2, Wc, bc, Wmu, bmu,
           Wsig, bsig, high, deterministic):
    N, NODE = x.shape
    E = edge_index.shape[1]
    EA = edge_attr.shape[1]
    HID = W2.shape[0]
    NF = high.shape[0]
    NN = 100
    G = N // NN

    ii = edge_index[0]
    jj = edge_index[1]

    epw = E // NW
    chunk = 2000
    mesh = plsc.VectorSubcoreMesh(core_axis_name="c", subcore_axis_name="s")
    sc_params = pltpu.CompilerParams(use_tc_tiling_on_sc=False)

    gather = pl.kernel(
        functools.partial(_gather_body, epw=epw, chunk=chunk),
        out_type=[jax.ShapeDtypeStruct((E, NODE), _F32),
                  jax.ShapeDtypeStruct((E, NODE), _F32)],
        mesh=mesh,
        scratch_types=[pltpu.VMEM((chunk,), jnp.int32),
                       pltpu.VMEM((chunk,), jnp.int32),
                       pltpu.VMEM((chunk, NODE), _F32),
                       pltpu.VMEM((chunk, NODE), _F32),
                       pltpu.SemaphoreType.DMA,
                       pltpu.SemaphoreType.DMA],
        compiler_params=sc_params,
    )
    xi, xj = gather(x, ii, jj)

    P = 8
    eye8 = jnp.eye(P, dtype=_F32)
    w1a = jnp.kron(eye8, W1[:, :NODE].T)
    w1b = jnp.kron(eye8, W1[:, NODE:2 * NODE].T)
    w1c = jnp.kron(eye8, W1[:, 2 * NODE:].T)
    w2t = jnp.kron(eye8, W2.T)
    b1r = jnp.tile(b1, P).reshape(1, P * HID)
    b2r = jnp.tile(b2, P).reshape(1, P * HID)
    xi_p = xi.reshape(E // P, P * NODE)
    xj_p = xj.reshape(E // P, P * NODE)
    ea_p = edge_attr.reshape(E // P, P * EA)
    BR = 2000
    ER = E // P
    msg_p = pl.pallas_call(
        _mlp_body,
        grid=(ER // BR,),
        in_specs=[
            pl.BlockSpec((BR, P * NODE), lambda e: (e, 0)),
            pl.BlockSpec((BR, P * NODE), lambda e: (e, 0)),
            pl.BlockSpec((BR, P * EA), lambda e: (e, 0)),
            pl.BlockSpec((P * NODE, P * HID), lambda e: (0, 0)),
            pl.BlockSpec((P * NODE, P * HID), lambda e: (0, 0)),
            pl.BlockSpec((P * EA, P * HID), lambda e: (0, 0)),
            pl.BlockSpec((1, P * HID), lambda e: (0, 0)),
            pl.BlockSpec((P * HID, P * HID), lambda e: (0, 0)),
            pl.BlockSpec((1, P * HID), lambda e: (0, 0)),
        ],
        out_specs=pl.BlockSpec((2 * BR, 128), lambda e: (e, 0)),
        out_shape=jax.ShapeDtypeStruct((E // 4, 128), _F32),
    )(xi_p, xj_p, ea_p, w1a, w1b, w1c, b1r, w2t, b2r)
    msg = msg_p.reshape(E, HID)
    ii_s = ii.reshape(E // 64, 8, 2, 4).transpose(0, 2, 1, 3).reshape(E)

    schunk = 400
    zeros = jnp.zeros((N, HID), _F32)
    scatter = pl.kernel(
        functools.partial(_scatter_body, n=N, hid=HID, epw=epw, chunk=schunk),
        out_type=jax.ShapeDtypeStruct((NC * N, HID), _F32),
        mesh=mesh,
        scratch_types=[pltpu.VMEM((schunk,), jnp.int32),
                       pltpu.VMEM((schunk, HID), _F32),
                       pltpu.VMEM_SHARED((N, HID), _F32)],
        compiler_params=sc_params,
    )
    part = scatter(zeros, msg, ii_s)

    xr = x.reshape(G, NN * NODE)
    pr = part.reshape(NC, G, NN * HID)
    eye = jnp.eye(NN, dtype=_F32)
    sel = (jnp.arange(NN)[:, None] == (NN - NF + jnp.arange(NF))[None, :])
    sel = sel.astype(_F32)
    wcx, wch = Wc[0, :NODE], Wc[0, NODE:]
    wmx, wmh = Wmu[0, :NODE], Wmu[0, NODE:]
    wsx, wsh = Wsig[0, :NODE], Wsig[0, NODE:]
    Wbx = jnp.kron(eye, wcx.reshape(NODE, 1))
    Wbh = jnp.kron(eye, wch.reshape(HID, 1))
    Wmx = jnp.kron(sel, wmx.reshape(NODE, 1))
    Wmh = jnp.kron(sel, wmh.reshape(HID, 1))
    Wsx = jnp.kron(sel, wsx.reshape(NODE, 1))
    Wsh = jnp.kron(sel, wsh.reshape(HID, 1))
    bcr = bc.reshape(1, 1)
    bmur = bmu.reshape(1, 1)
    bsigr = bsig.reshape(1, 1)
    highr = high.reshape(1, NF)

    inv, ordv = pl.pallas_call(
        _node_body,
        out_shape=[jax.ShapeDtypeStruct((G, NN), _F32),
                   jax.ShapeDtypeStruct((G, NF), _F32)],
        compiler_params=pltpu.CompilerParams(vmem_limit_bytes=63 << 20),
    )(xr, pr, Wbx, Wbh, Wmx, Wmh, Wsx, Wsh, bcr, bmur, bsigr, highr)

    return (inv, ordv)


# speedup vs baseline: 1.1307x; 1.1307x over previous
import functools

import jax
import jax.numpy as jnp
from jax import lax
from jax.experimental import pallas as pl
from jax.experimental.pallas import tpu as pltpu
from jax.experimental.pallas import tpu_sc as plsc

NC, NS = 2, 16
NW = NC * NS
_F32 = jnp.float32
_HI = lax.Precision.HIGHEST


def _gather_body(x_hbm, ii_hbm, jj_hbm, xi_hbm, xj_hbm,
                 iv, jv, ri, rj, s1, s2, *, epw, chunk):
    c = lax.axis_index("c")
    s = lax.axis_index("s")
    wid = s * NC + c

    def body(k, carry):
        base = wid * epw + k * chunk
        a = pltpu.async_copy(ii_hbm.at[pl.ds(base, chunk)], iv, s1)
        b = pltpu.async_copy(jj_hbm.at[pl.ds(base, chunk)], jv, s2)
        a.wait()
        b.wait()
        g1 = pltpu.async_copy(x_hbm.at[iv], ri, s1)
        g2 = pltpu.async_copy(x_hbm.at[jv], rj, s2)
        g1.wait()
        g2.wait()
        w1 = pltpu.async_copy(ri, xi_hbm.at[pl.ds(base, chunk)], s1)
        w2 = pltpu.async_copy(rj, xj_hbm.at[pl.ds(base, chunk)], s2)
        w1.wait()
        w2.wait()
        return carry

    lax.fori_loop(0, epw // chunk, body, 0)


def _scatter_body(z_hbm, mA_hbm, mB_hbm, iA_hbm, iB_hbm, out_hbm,
                  iv, mv, acc, *, n, hid, eh, chunk):
    c = lax.axis_index("c")
    s = lax.axis_index("s")
    wid = s * NC + c
    stripe = n // NS

    pltpu.sync_copy(z_hbm.at[pl.ds(s * stripe, stripe)],
                    acc.at[pl.ds(s * stripe, stripe)])
    plsc.subcore_barrier()

    def run(m_hbm, i_hbm):
        def body(k, carry):
            base = wid * eh + k * chunk
            pltpu.sync_copy(i_hbm.at[pl.ds(base, chunk)], iv)
            pltpu.sync_copy(m_hbm.at[pl.ds(base, chunk)], mv)
            pltpu.sync_copy(mv, acc.at[iv], add=True)
            return carry

        lax.fori_loop(0, eh // chunk, body, 0)

    run(mA_hbm, iA_hbm)
    run(mB_hbm, iB_hbm)
    plsc.subcore_barrier()
    pltpu.sync_copy(acc.at[pl.ds(s * stripe, stripe)],
                    out_hbm.at[pl.ds(c * n + s * stripe, stripe)])


def _mlp_body(xi, xj, eA, eB, w1a, w1b, b1, w2t4, b2, outA, outB):
    s = (jnp.dot(xi[...], w1a[...], preferred_element_type=_F32)
         + jnp.dot(xj[...], w1b[...], preferred_element_type=_F32)
         + b1[...])
    hA = jnp.maximum(s[:, :128] + eA[...], 0.0)
    hB = jnp.maximum(s[:, 128:] + eB[...], 0.0)
    outA[...] = jnp.dot(hA, w2t4[...], preferred_element_type=_F32) + b2[...]
    outB[...] = jnp.dot(hB, w2t4[...], preferred_element_type=_F32) + b2[...]


def _eaproj_body(ea_hbm, w_hbm, outA_hbm, outB_hbm, eabuf, bufA, bufB, wbuf,
                 *, nrows, ct):
    c = lax.axis_index("c")
    s = lax.axis_index("s")
    wid = s * NC + c
    pltpu.sync_copy(w_hbm, wbuf)
    wv = []
    for k in range(4):
        wv.append(wbuf[k, pl.ds(0, 16)])
        wv.append(wbuf[k, pl.ds(16, 16)])
    nq = nrows // ct
    nchunks = (nq + NW - 1) // NW

    def chunk(i, carry):
        q = wid + i * NW

        @pl.when(q < nq)
        def _():
            pltpu.sync_copy(ea_hbm.at[pl.ds(q * ct, ct)], eabuf)

            def pair(g, cin):
                v = cin
                t = g // 8
                lb = (g % 8) * 16
                ev = [eabuf[t, k, pl.ds(lb, 16)] for k in range(4)]
                r8 = g * 8
                for u in range(16):
                    e0, e1 = ev[0][u], ev[1][u]
                    e2, e3 = ev[2][u], ev[3][u]
                    flo = e0 * v[0] + e1 * v[2] + e2 * v[4] + e3 * v[6]
                    fhi = e0 * v[1] + e1 * v[3] + e2 * v[5] + e3 * v[7]
                    r = r8 + 4 * (u // 8) + (u % 4)
                    if (u // 4) % 2 == 0:
                        bufA[r, pl.ds(0, 16)] = flo
                        bufA[r, pl.ds(16, 16)] = fhi
                    else:
                        bufB[r, pl.ds(0, 16)] = flo
                        bufB[r, pl.ds(16, 16)] = fhi
                return v

            lax.fori_loop(0, ct * 8, pair, tuple(wv))
            half = ct * 64
            pltpu.sync_copy(bufA, outA_hbm.at[pl.ds(q * half, half)])
            pltpu.sync_copy(bufB, outB_hbm.at[pl.ds(q * half, half)])

        return carry

    lax.fori_loop(0, nchunks, chunk, 0)


def _softplus(z):
    return jnp.maximum(z, 0.0) + jnp.log(1.0 + jnp.exp(-jnp.abs(z)))


def _node_body(xr, pr, wbx, wbh, wmx, wmh, wsx, wsh,
               bc, bmu, bsig, highr, inv_out, ord_out):
    xpp = pr[0] + pr[1]
    xv = xr[...]
    zc = (jnp.dot(xv, wbx[...], preferred_element_type=_F32, precision=_HI)
          + jnp.dot(xpp, wbh[...], preferred_element_type=_F32, precision=_HI)
          + bc[...] + 1e-10)
    conc = _softplus(zc)
    total = jnp.sum(conc)
    inv_out[...] = conc / (total + 1e-20)
    zmu = (jnp.dot(xv, wmx[...], preferred_element_type=_F32, precision=_HI)
           + jnp.dot(xpp, wmh[...], preferred_element_type=_F32, precision=_HI)
           + bmu[...] + 1e-20)
    a = _softplus(zmu) + 1e-20
    zsg = (jnp.dot(xv, wsx[...], preferred_element_type=_F32, precision=_HI)
           + jnp.dot(xpp, wsh[...], preferred_element_type=_F32, precision=_HI)
           + bsig[...] + 1e-20)
    b = _softplus(zsg) + 1e-20
    ord_out[...] = a / (a + b) * highr[...]


def kernel(x, edge_index, edge_attr, W1, b1, W2, b2, Wc, bc, Wmu, bmu,
           Wsig, bsig, high, deterministic):
    N, NODE = x.shape
    E = edge_index.shape[1]
    EA = edge_attr.shape[1]
    HID = W2.shape[0]
    NF = high.shape[0]
    NN = 100
    G = N // NN

    ii = edge_index[0]
    jj = edge_index[1]

    epw = E // NW
    chunk = 2000
    mesh = plsc.VectorSubcoreMesh(core_axis_name="c", subcore_axis_name="s")
    sc_params = pltpu.CompilerParams(use_tc_tiling_on_sc=False)

    gather = pl.kernel(
        functools.partial(_gather_body, epw=epw, chunk=chunk),
        out_type=[jax.ShapeDtypeStruct((E, NODE), _F32),
                  jax.ShapeDtypeStruct((E, NODE), _F32)],
        mesh=mesh,
        scratch_types=[pltpu.VMEM((chunk,), jnp.int32),
                       pltpu.VMEM((chunk,), jnp.int32),
                       pltpu.VMEM((chunk, NODE), _F32),
                       pltpu.VMEM((chunk, NODE), _F32),
                       pltpu.SemaphoreType.DMA,
                       pltpu.SemaphoreType.DMA],
        compiler_params=sc_params,
    )
    xi, xj = gather(x, ii, jj)

    eaV = edge_attr.reshape(E // 128, 128, EA).transpose(0, 2, 1)
    w1cT = W1[:, 2 * NODE:].T
    CT = 25
    eaproj = pl.kernel(
        functools.partial(_eaproj_body, nrows=E // 128, ct=CT),
        out_type=[jax.ShapeDtypeStruct((E // 2, HID), _F32),
                  jax.ShapeDtypeStruct((E // 2, HID), _F32)],
        mesh=mesh,
        scratch_types=[pltpu.VMEM((CT, EA, 128), _F32),
                       pltpu.VMEM((CT * 64, HID), _F32),
                       pltpu.VMEM((CT * 64, HID), _F32),
                       pltpu.VMEM((EA, HID), _F32)],
        compiler_params=sc_params,
    )
    epA, epB = eaproj(eaV, w1cT)

    P = 8
    eye8 = jnp.eye(P, dtype=_F32)
    w1a = jnp.kron(eye8, W1[:, :NODE].T)
    w1b = jnp.kron(eye8, W1[:, NODE:2 * NODE].T)
    w2t4 = jnp.kron(jnp.eye(4, dtype=_F32), W2.T)
    b1r = jnp.tile(b1, P).reshape(1, P * HID)
    b2r4 = jnp.tile(b2, 4).reshape(1, 4 * HID)
    xi_p = xi.reshape(E // P, P * NODE)
    xj_p = xj.reshape(E // P, P * NODE)
    epA_p = epA.reshape(E // P, 128)
    epB_p = epB.reshape(E // P, 128)
    BR = 2000
    ER = E // P
    msgA_p, msgB_p = pl.pallas_call(
        _mlp_body,
        grid=(ER // BR,),
        in_specs=[
            pl.BlockSpec((BR, P * NODE), lambda e: (e, 0)),
            pl.BlockSpec((BR, P * NODE), lambda e: (e, 0)),
            pl.BlockSpec((BR, 128), lambda e: (e, 0)),
            pl.BlockSpec((BR, 128), lambda e: (e, 0)),
            pl.BlockSpec((P * NODE, P * HID), lambda e: (0, 0)),
            pl.BlockSpec((P * NODE, P * HID), lambda e: (0, 0)),
            pl.BlockSpec((1, P * HID), lambda e: (0, 0)),
            pl.BlockSpec((4 * HID, 4 * HID), lambda e: (0, 0)),
            pl.BlockSpec((1, 4 * HID), lambda e: (0, 0)),
        ],
        out_specs=[pl.BlockSpec((BR, 128), lambda e: (e, 0)),
                   pl.BlockSpec((BR, 128), lambda e: (e, 0))],
        out_shape=[jax.ShapeDtypeStruct((ER, 128), _F32),
                   jax.ShapeDtypeStruct((ER, 128), _F32)],
    )(xi_p, xj_p, epA_p, epB_p, w1a, w1b, b1r, w2t4, b2r4)
    msgA = msgA_p.reshape(E // 2, HID)
    msgB = msgB_p.reshape(E // 2, HID)
    iiA = ii.reshape(E // P, 2, 4)[:, 0, :].reshape(E // 2)
    iiB = ii.reshape(E // P, 2, 4)[:, 1, :].reshape(E // 2)

    schunk = 200
    zeros = jnp.zeros((N, HID), _F32)
    scatter = pl.kernel(
        functools.partial(_scatter_body, n=N, hid=HID, eh=E // 2 // NW,
                          chunk=schunk),
        out_type=jax.ShapeDtypeStruct((NC * N, HID), _F32),
        mesh=mesh,
        scratch_types=[pltpu.VMEM((schunk,), jnp.int32),
                       pltpu.VMEM((schunk, HID), _F32),
                       pltpu.VMEM_SHARED((N, HID), _F32)],
        compiler_params=sc_params,
    )
    part = scatter(zeros, msgA, msgB, iiA, iiB)

    xr = x.reshape(G, NN * NODE)
    pr = part.reshape(NC, G, NN * HID)
    eye = jnp.eye(NN, dtype=_F32)
    sel = (jnp.arange(NN)[:, None] == (NN - NF + jnp.arange(NF))[None, :])
    sel = sel.astype(_F32)
    wcx, wch = Wc[0, :NODE], Wc[0, NODE:]
    wmx, wmh = Wmu[0, :NODE], Wmu[0, NODE:]
    wsx, wsh = Wsig[0, :NODE], Wsig[0, NODE:]
    Wbx = jnp.kron(eye, wcx.reshape(NODE, 1))
    Wbh = jnp.kron(eye, wch.reshape(HID, 1))
    Wmx = jnp.kron(sel, wmx.reshape(NODE, 1))
    Wmh = jnp.kron(sel, wmh.reshape(HID, 1))
    Wsx = jnp.kron(sel, wsx.reshape(NODE, 1))
    Wsh = jnp.kron(sel, wsh.reshape(HID, 1))
    bcr = bc.reshape(1, 1)
    bmur = bmu.reshape(1, 1)
    bsigr = bsig.reshape(1, 1)
    highr = high.reshape(1, NF)

    inv, ordv = pl.pallas_call(
        _node_body,
        out_shape=[jax.ShapeDtypeStruct((G, NN), _F32),
                   jax.ShapeDtypeStruct((G, NF), _F32)],
        compiler_params=pltpu.CompilerParams(vmem_limit_bytes=63 << 20),
    )(xr, pr, Wbx, Wbh, Wmx, Wmh, Wsx, Wsh, bcr, bmur, bsigr, highr)

    return (inv, ordv)
